# Initial kernel scaffold; baseline (speedup 1.0000x reference)
import jax
import jax.numpy as jnp
from jax.experimental import pallas as pl


def kernel(x, edge_index, batch, edge_weight, W1, b1, W2, b2, W3, b3, W_lin1, b_lin1, W_lin2, b_lin2):
    raise NotImplementedError("write your pallas kernel here")


# speedup vs baseline: 13.6685x; 13.6685x over previous
import functools

import jax
import jax.numpy as jnp
from jax import lax
from jax.experimental import pallas as pl
from jax.experimental.pallas import tpu as pltpu
from jax.experimental.pallas import tpu_sc as plsc

N = 10000
E = 320000
D = 128
G = 32

NP = 10240
TN = 1024
NT = NP // TN
K = 128
EPC = E // 2
NCH = EPC // K
RPT = NP // 16

_mesh = plsc.VectorSubcoreMesh(core_axis_name="c", subcore_axis_name="s")


def _deg_body(dst_hbm, out_hbm, dstv, onesv, zbuf, deg_sh):
    c = lax.axis_index("c")
    s = lax.axis_index("s")
    for j in range(RPT // 16):
        zbuf[pl.ds(j * 16, 16)] = jnp.zeros((16,), jnp.float32)
    for j in range(K // 16):
        onesv[pl.ds(j * 16, 16)] = jnp.ones((16,), jnp.float32)
    pltpu.sync_copy(zbuf, deg_sh.at[pl.ds(s * RPT, RPT)])
    plsc.subcore_barrier()

    def body(i, carry):
        j = s + i * 16

        @pl.when(j < NCH)
        def _():
            off = c * EPC + j * K
            pltpu.sync_copy(dst_hbm.at[pl.ds(off, K)], dstv)
            pltpu.sync_copy(onesv, deg_sh.at[dstv], add=True)

        return carry

    lax.fori_loop(0, (NCH + 15) // 16, body, 0)
    plsc.subcore_barrier()
    pltpu.sync_copy(deg_sh.at[pl.ds(s * RPT, RPT)], out_hbm.at[c, pl.ds(s * RPT, RPT)])


_deg_call = pl.kernel(
    _deg_body,
    out_type=jax.ShapeDtypeStruct((2, NP), jnp.float32),
    mesh=_mesh,
    scratch_types=[
        pltpu.VMEM((K,), jnp.int32),
        pltpu.VMEM((K,), jnp.float32),
        pltpu.VMEM((RPT,), jnp.float32),
        pltpu.VMEM_SHARED((NP,), jnp.float32),
    ],
)


def _agg_body(y_hbm, yh_hbm, src_hbm, dst_hbm, p_hbm, srcv, dstv, rows, acc_sh, sem):
    c = lax.axis_index("c")
    s = lax.axis_index("s")
    r0 = s * RPT
    pltpu.sync_copy(yh_hbm.at[pl.ds(r0, RPT)], acc_sh.at[pl.ds(r0, RPT)])
    plsc.subcore_barrier()

    def body(i, carry):
        j = s + i * 16

        @pl.when(j < NCH)
        def _():
            off = c * EPC + j * K
            pltpu.sync_copy(src_hbm.at[pl.ds(off, K)], srcv)
            pltpu.async_copy(y_hbm.at[srcv], rows, sem).wait()
            pltpu.sync_copy(dst_hbm.at[pl.ds(off, K)], dstv)
            pltpu.sync_copy(rows, acc_sh.at[dstv], add=True)

        return carry

    lax.fori_loop(0, (NCH + 15) // 16, body, 0)
    plsc.subcore_barrier()
    pltpu.sync_copy(acc_sh.at[pl.ds(r0, RPT)], p_hbm.at[c, pl.ds(r0, RPT)])


_agg_call = pl.kernel(
    _agg_body,
    out_type=jax.ShapeDtypeStruct((2, NP, D), jnp.float32),
    mesh=_mesh,
    scratch_types=[
        pltpu.VMEM((K,), jnp.int32),
        pltpu.VMEM((K,), jnp.int32),
        pltpu.VMEM((K, D), jnp.float32),
        pltpu.VMEM_SHARED((NP, D), jnp.float32),
        pltpu.SemaphoreType.DMA,
    ],
)


def _gelu(x):
    return 0.5 * x * (1.0 + lax.erf(x * 0.7071067811865476))


def _dinv_body(deg_ref, out_ref):
    out_ref[...] = lax.rsqrt(deg_ref[0:1, :] + deg_ref[1:2, :] + 1.0)


_dinv_call = pl.pallas_call(
    _dinv_body,
    out_shape=jax.ShapeDtypeStruct((1, NP), jnp.float32),
)


def _prep_body(x_ref, w_ref, dinv_ref, y_ref, yh_ref):
    y = dinv_ref[...] * jnp.dot(x_ref[...], w_ref[...],
                                preferred_element_type=jnp.float32)
    y_ref[...] = y
    yh_ref[...] = 0.5 * y


_prep_call = pl.pallas_call(
    _prep_body,
    grid=(NT,),
    in_specs=[
        pl.BlockSpec((TN, D), lambda i: (i, 0)),
        pl.BlockSpec((D, D), lambda i: (0, 0)),
        pl.BlockSpec((TN, 1), lambda i: (i, 0)),
    ],
    out_specs=[
        pl.BlockSpec((TN, D), lambda i: (i, 0)),
        pl.BlockSpec((TN, D), lambda i: (i, 0)),
    ],
    out_shape=[
        jax.ShapeDtypeStruct((NP, D), jnp.float32),
        jax.ShapeDtypeStruct((NP, D), jnp.float32),
    ],
)


def _mid_body(p_ref, dinv_ref, b_ref, w_ref, y_ref, yh_ref):
    dv = dinv_ref[...]
    h = _gelu(dv * (p_ref[0] + p_ref[1]) + b_ref[...])
    y = dv * jnp.dot(h, w_ref[...], preferred_element_type=jnp.float32)
    y_ref[...] = y
    yh_ref[...] = 0.5 * y


_mid_call = pl.pallas_call(
    _mid_body,
    grid=(NT,),
    in_specs=[
        pl.BlockSpec((2, TN, D), lambda i: (0, i, 0)),
        pl.BlockSpec((TN, 1), lambda i: (i, 0)),
        pl.BlockSpec((1, D), lambda i: (0, 0)),
        pl.BlockSpec((D, D), lambda i: (0, 0)),
    ],
    out_specs=[
        pl.BlockSpec((TN, D), lambda i: (i, 0)),
        pl.BlockSpec((TN, D), lambda i: (i, 0)),
    ],
    out_shape=[
        jax.ShapeDtypeStruct((NP, D), jnp.float32),
        jax.ShapeDtypeStruct((NP, D), jnp.float32),
    ],
)


def _final_body(p_ref, dinv_ref, b3_ref, batch_ref, wl1_ref, bl1_ref,
                wl2_ref, bl2_ref, out_ref, acc_s, acc_c):
    i = pl.program_id(0)
    dv = dinv_ref[...]
    h = _gelu(dv * (p_ref[0] + p_ref[1]) + b3_ref[...])
    bt = batch_ref[0]
    gids = lax.broadcasted_iota(jnp.int32, (G, TN), 0)
    oh = (bt == gids).astype(jnp.float32)

    @pl.when(i == 0)
    def _():
        acc_s[...] = jnp.zeros_like(acc_s)
        acc_c[...] = jnp.zeros_like(acc_c)

    acc_s[...] += jnp.dot(oh, h, preferred_element_type=jnp.float32)
    acc_c[...] += jnp.sum(oh, axis=1, keepdims=True)

    @pl.when(i == pl.num_programs(0) - 1)
    def _():
        pooled = acc_s[...] / jnp.maximum(acc_c[...], 1.0)
        t = jnp.dot(pooled, wl1_ref[...],
                    preferred_element_type=jnp.float32) + bl1_ref[...]
        t = jnp.where(t > 0, t, jnp.exp(jnp.minimum(t, 0.0)) - 1.0)
        out_ref[...] = jnp.dot(t, wl2_ref[...],
                               preferred_element_type=jnp.float32) + bl2_ref[...]


_final_call = pl.pallas_call(
    _final_body,
    grid=(NT,),
    in_specs=[
        pl.BlockSpec((2, TN, D), lambda i: (0, i, 0)),
        pl.BlockSpec((TN, 1), lambda i: (i, 0)),
        pl.BlockSpec((1, D), lambda i: (0, 0)),
        pl.BlockSpec((1, 1, TN), lambda i: (i, 0, 0)),
        pl.BlockSpec((D, D // 2), lambda i: (0, 0)),
        pl.BlockSpec((1, D // 2), lambda i: (0, 0)),
        pl.BlockSpec((D // 2, 1), lambda i: (0, 0)),
        pl.BlockSpec((1, 1), lambda i: (0, 0)),
    ],
    out_specs=pl.BlockSpec((G, 1), lambda i: (0, 0)),
    out_shape=jax.ShapeDtypeStruct((G, 1), jnp.float32),
    scratch_shapes=[
        pltpu.VMEM((G, D), jnp.float32),
        pltpu.VMEM((G, 1), jnp.float32),
    ],
)


def kernel(x, edge_index, batch, edge_weight, W1, b1, W2, b2, W3, b3,
           W_lin1, b_lin1, W_lin2, b_lin2):
    src = edge_index[0].astype(jnp.int32)
    dst = edge_index[1].astype(jnp.int32)
    x_p = jnp.pad(x, ((0, NP - N), (0, 0)))
    batch_p = jnp.pad(batch.astype(jnp.int32), (0, NP - N),
                      constant_values=G).reshape(NT, 1, TN)

    deg = _deg_call(dst)
    dinv = _dinv_call(deg).reshape(NP, 1)

    y, yh = _prep_call(x_p, W1, dinv)
    p = _agg_call(y, yh, src, dst)
    y, yh = _mid_call(p, dinv, b1.reshape(1, D), W2)
    p = _agg_call(y, yh, src, dst)
    y, yh = _mid_call(p, dinv, b2.reshape(1, D), W3)
    p = _agg_call(y, yh, src, dst)
    return _final_call(p, dinv, b3.reshape(1, D), batch_p, W_lin1,
                       b_lin1.reshape(1, D // 2), W_lin2,
                       b_lin2.reshape(1, 1))
